# Initial kernel scaffold; baseline (speedup 1.0000x reference)
import jax
import jax.numpy as jnp
from jax.experimental import pallas as pl


def kernel(scores, bbox_deltas, im_info):
    raise NotImplementedError("write your pallas kernel here")


# speedup vs baseline: 103.5837x; 103.5837x over previous
import numpy as np
import jax
import jax.numpy as jnp
from jax.experimental import pallas as pl
from jax.experimental.pallas import tpu as pltpu

_NUM_A = 9
_PRE = 6000
_POST = 300
_T = 0.7
_BLK = 128
_NBLK = 47
_PAD = _NBLK * _BLK
_KCAP = 304


def _anchor_params():
    base = np.array([1.0, 1.0, 16.0, 16.0]) - 1
    w = base[2] - base[0] + 1
    h = base[3] - base[1] + 1
    x_ctr = base[0] + 0.5 * (w - 1)
    y_ctr = base[1] + 0.5 * (h - 1)
    size = w * h
    ratio_anchors = []
    for r in (0.5, 1.0, 2.0):
        ws = np.round(np.sqrt(size / r))
        hs = np.round(ws * r)
        ratio_anchors.append((x_ctr - 0.5 * (ws - 1), y_ctr - 0.5 * (hs - 1),
                              x_ctr + 0.5 * (ws - 1), y_ctr + 0.5 * (hs - 1)))
    rows = []
    for (x1, y1, x2, y2) in ratio_anchors:
        aw = x2 - x1 + 1
        ah = y2 - y1 + 1
        xc = x1 + 0.5 * (aw - 1)
        yc = y1 + 0.5 * (ah - 1)
        for s in (8.0, 16.0, 32.0):
            ws = aw * s
            hs = ah * s
            rows.append((xc - 0.5 * (ws - 1), yc - 0.5 * (hs - 1),
                         xc + 0.5 * (ws - 1), yc + 0.5 * (hs - 1)))
    tab = np.array(rows, dtype=np.float32)
    ax1 = tab[:, 0]
    ay1 = tab[:, 1]
    aw = tab[:, 2] - tab[:, 0] + np.float32(1.0)
    ah = tab[:, 3] - tab[:, 1] + np.float32(1.0)
    return ([float(v) for v in ax1], [float(v) for v in ay1],
            [float(v) for v in aw], [float(v) for v in ah])

_AX1, _AY1, _AW, _AH = _anchor_params()


def _nms_kernel(info_ref, dx_ref, dy_ref, dw_ref, dh_ref, nf_ref,
                out_ref, iou_s, kept_s, blk_s):
    b = pl.program_id(0)
    hm1 = info_ref[b, 0] - 1.0
    wm1 = info_ref[b, 1] - 1.0
    lanei = jax.lax.broadcasted_iota(jnp.int32, (1, _BLK), 1)
    sub_k = jax.lax.broadcasted_iota(jnp.int32, (_KCAP, 1), 0)

    def block_body(state):
        blk, kc = state
        dx = dx_ref[0, pl.ds(blk, 1), :]
        dy = dy_ref[0, pl.ds(blk, 1), :]
        dw = dw_ref[0, pl.ds(blk, 1), :]
        dh = dh_ref[0, pl.ds(blk, 1), :]
        nf = nf_ref[0, pl.ds(blk, 1), :]

        k = jnp.floor(nf / 9.0)
        a = nf - 9.0 * k
        yq = jnp.floor(k / 64.0)
        xq = k - 64.0 * yq
        sx = xq * 16.0
        sy = yq * 16.0
        ax1 = jnp.zeros_like(nf)
        ay1 = jnp.zeros_like(nf)
        aw = jnp.zeros_like(nf)
        ah = jnp.zeros_like(nf)
        for j in range(_NUM_A):
            sel = a == float(j)
            ax1 = jnp.where(sel, _AX1[j], ax1)
            ay1 = jnp.where(sel, _AY1[j], ay1)
            aw = jnp.where(sel, _AW[j], aw)
            ah = jnp.where(sel, _AH[j], ah)
        x1b = ax1 + sx
        y1b = ay1 + sy
        ctrx = x1b + 0.5 * aw
        ctry = y1b + 0.5 * ah
        pcx = dx * aw + ctrx
        pcy = dy * ah + ctry
        pw = jnp.exp(dw) * aw
        ph = jnp.exp(dh) * ah
        px1 = jnp.clip(pcx - 0.5 * pw, 0.0, wm1)
        py1 = jnp.clip(pcy - 0.5 * ph, 0.0, hm1)
        px2 = jnp.clip(pcx + 0.5 * pw, 0.0, wm1)
        py2 = jnp.clip(pcy + 0.5 * ph, 0.0, hm1)
        area = (px2 - px1 + 1.0) * (py2 - py1 + 1.0)

        validf = jnp.where(blk * _BLK + lanei < _PRE, 0.0, 1.0)

        kx1 = kept_s[:, 0:1]
        ky1 = kept_s[:, 1:2]
        kx2 = kept_s[:, 2:3]
        ky2 = kept_s[:, 3:4]
        kar = kept_s[:, 4:5]
        xx1 = jnp.maximum(kx1, px1)
        yy1 = jnp.maximum(ky1, py1)
        xx2 = jnp.minimum(kx2, px2)
        yy2 = jnp.minimum(ky2, py2)
        iw = jnp.maximum(0.0, xx2 - xx1 + 1.0)
        ih = jnp.maximum(0.0, yy2 - yy1 + 1.0)
        inter = iw * ih
        iouc = inter / (kar + area - inter)
        hit = jnp.where((iouc > _T) & (sub_k < kc), 1.0, 0.0)
        supc = jnp.max(hit, axis=0, keepdims=True)
        sup0 = jnp.maximum(supc, validf)

        tx1 = jnp.transpose(px1)
        ty1 = jnp.transpose(py1)
        tx2 = jnp.transpose(px2)
        ty2 = jnp.transpose(py2)
        tar = jnp.transpose(area)
        blk_s[:, 0:1] = tx1
        blk_s[:, 1:2] = ty1
        blk_s[:, 2:3] = tx2
        blk_s[:, 3:4] = ty2
        blk_s[:, 4:5] = tar
        bx1 = jnp.maximum(tx1, px1)
        by1 = jnp.maximum(ty1, py1)
        bx2 = jnp.minimum(tx2, px2)
        by2 = jnp.minimum(ty2, py2)
        bw = jnp.maximum(0.0, bx2 - bx1 + 1.0)
        bh = jnp.maximum(0.0, by2 - by1 + 1.0)
        binter = bw * bh
        iou_s[:, :] = binter / (tar + area - binter)

        def scond(st):
            i, kc2, _ = st
            return (i < _BLK) & (kc2 < _POST)

        def sbody(st):
            i, kc2, sup = st
            s_i = jnp.sum(jnp.where(lanei == i, sup, 0.0))
            keep = s_i < 0.5

            @pl.when(keep)
            def _():
                kept_s[pl.ds(kc2, 1), 0:5] = blk_s[pl.ds(i, 1), 0:5]

            row = iou_s[pl.ds(i, 1), :]
            newsup = jnp.maximum(
                sup, jnp.where((row > _T) & (lanei > i), 1.0, 0.0))
            sup = jnp.where(keep, newsup, sup)
            kc2 = kc2 + jnp.where(keep, jnp.int32(1), jnp.int32(0))
            return i + jnp.int32(1), kc2, sup

        _, kc_new, _ = jax.lax.while_loop(
            scond, sbody, (jnp.int32(0), kc, sup0))
        return blk + jnp.int32(1), kc_new

    def ocond(state):
        blk, kc = state
        return (blk < _NBLK) & (kc < _POST)

    _, kc = jax.lax.while_loop(ocond, block_body, (jnp.int32(0), jnp.int32(0)))

    bf = jnp.zeros((), jnp.float32) + b.astype(jnp.float32)
    bcol = jnp.zeros((_KCAP, 1), jnp.float32) + bf
    coords = jnp.where(sub_k < kc, kept_s[:, 0:4], 0.0)
    out_ref[0, :, :] = jnp.concatenate(
        [bcol, coords, jnp.zeros((_KCAP, 123), jnp.float32)], axis=1)


def kernel(scores, bbox_deltas, im_info):
    B = scores.shape[0]
    fg = scores[:, _NUM_A:, :, :]
    sc = jnp.transpose(fg, (0, 2, 3, 1)).reshape(B, -1)
    deltas = jnp.transpose(bbox_deltas, (0, 2, 3, 1)).reshape(B, -1, 4)
    _, idx = jax.lax.top_k(sc, _PRE)
    d = jnp.take_along_axis(deltas, idx[..., None], axis=1)
    pad = _PAD - _PRE
    d = jnp.pad(d, ((0, 0), (0, pad), (0, 0)))
    nf = jnp.pad(idx, ((0, 0), (0, pad))).astype(jnp.float32)
    dx = d[..., 0].reshape(B, _NBLK, _BLK)
    dy = d[..., 1].reshape(B, _NBLK, _BLK)
    dw = d[..., 2].reshape(B, _NBLK, _BLK)
    dh = d[..., 3].reshape(B, _NBLK, _BLK)
    nf = nf.reshape(B, _NBLK, _BLK)

    blkspec = pl.BlockSpec((1, _NBLK, _BLK), lambda b: (b, 0, 0))
    out = pl.pallas_call(
        _nms_kernel,
        grid=(B,),
        in_specs=[
            pl.BlockSpec(memory_space=pltpu.SMEM),
            blkspec, blkspec, blkspec, blkspec, blkspec,
        ],
        out_specs=pl.BlockSpec((1, _KCAP, 128), lambda b: (b, 0, 0)),
        out_shape=jax.ShapeDtypeStruct((B, _KCAP, 128), jnp.float32),
        scratch_shapes=[
            pltpu.VMEM((_BLK, _BLK), jnp.float32),
            pltpu.VMEM((_KCAP, 8), jnp.float32),
            pltpu.VMEM((_BLK, 8), jnp.float32),
        ],
    )(im_info, dx, dy, dw, dh, nf)
    return out[:, :_POST, :5]
